# Initial kernel scaffold; baseline (speedup 1.0000x reference)
import jax
import jax.numpy as jnp
from jax.experimental import pallas as pl


def kernel(x, pos_table):
    raise NotImplementedError("write your pallas kernel here")


# speedup vs baseline: 1.6610x; 1.6610x over previous
import jax
import jax.numpy as jnp
from jax.experimental import pallas as pl


def _add_body(x_ref, pos_ref, o_ref):
    o_ref[...] = x_ref[...] + pos_ref[...]


def kernel(x, pos_table):
    B, T, D = x.shape
    bt = 256
    xf = x.reshape(B * T, D)
    nt = T // bt
    out = pl.pallas_call(
        _add_body,
        grid=(nt, B),
        in_specs=[
            pl.BlockSpec((bt, D), lambda t, b: (b * nt + t, 0)),
            pl.BlockSpec((bt, D), lambda t, b: (t, 0)),
        ],
        out_specs=pl.BlockSpec((bt, D), lambda t, b: (b * nt + t, 0)),
        out_shape=jax.ShapeDtypeStruct((B * T, D), x.dtype),
    )(xf, pos_table)
    return out.reshape(B, T, D)
